# Initial kernel scaffold; baseline (speedup 1.0000x reference)
import jax
import jax.numpy as jnp
from jax.experimental import pallas as pl


def kernel(users_feature, W1, root1, b1, W2, root2, b2, fc1_w, fc1_b, fc2_w, fc2_b, edge_index, edge_type, bill_id, user1_id, user2_id):
    raise NotImplementedError("write your pallas kernel here")


# speedup vs baseline: 2.8974x; 2.8974x over previous
import functools

import jax
import jax.numpy as jnp
from jax.experimental import pallas as pl
from jax.experimental.pallas import tpu as pltpu


def _proj_body(x_ref, wy_ref, wr_ref, y_ref, xr_ref):
    x = x_ref[...]
    y_ref[...] = x @ wy_ref[...]
    xr_ref[...] = x @ wr_ref[...]


def _proj(x, w_cat, root, rows_blk):
    n, k = x.shape
    f = w_cat.shape[1]
    h = root.shape[1]
    grid = (n // rows_blk,)
    return pl.pallas_call(
        _proj_body,
        grid=grid,
        in_specs=[
            pl.BlockSpec((rows_blk, k), lambda i: (i, 0)),
            pl.BlockSpec((k, f), lambda i: (0, 0)),
            pl.BlockSpec((k, h), lambda i: (0, 0)),
        ],
        out_specs=[
            pl.BlockSpec((rows_blk, f), lambda i: (i, 0)),
            pl.BlockSpec((rows_blk, h), lambda i: (i, 0)),
        ],
        out_shape=[
            jax.ShapeDtypeStruct((n, f), jnp.float32),
            jax.ShapeDtypeStruct((n, h), jnp.float32),
        ],
    )(x, w_cat, root)


def _combine_body(do_relu, r, h, xr_ref, acc_ref, cnt_ref, b_ref, o_ref):
    rows = xr_ref.shape[0]
    inv = 1.0 / jnp.maximum(cnt_ref[...], 1.0)
    acc = acc_ref[...].reshape(rows, r, h)
    agg = jnp.sum(acc * inv[:, :, None], axis=1)
    out = xr_ref[...] + b_ref[...] + agg
    if do_relu:
        out = jnp.maximum(out, 0.0)
    o_ref[...] = out


def _combine(xroot, acc, cnt, b, do_relu, rows_blk):
    n, h = xroot.shape
    r = cnt.shape[1]
    grid = (n // rows_blk,)
    return pl.pallas_call(
        functools.partial(_combine_body, do_relu, r, h),
        grid=grid,
        in_specs=[
            pl.BlockSpec((rows_blk, h), lambda i: (i, 0)),
            pl.BlockSpec((rows_blk, r * h), lambda i: (i, 0)),
            pl.BlockSpec((rows_blk, r), lambda i: (i, 0)),
            pl.BlockSpec((1, h), lambda i: (0, 0)),
        ],
        out_specs=pl.BlockSpec((rows_blk, h), lambda i: (i, 0)),
        out_shape=jax.ShapeDtypeStruct((n, h), jnp.float32),
    )(xroot, acc, cnt, b.reshape(1, h))


def _head_body(bill_ref, u1_ref, u2_ref, a1_ref, a2_ref, b1_ref,
               w2_ref, b2_ref, o_ref):
    bill = bill_ref[...]
    a1 = a1_ref[...]
    a2 = a2_ref[...]
    b1 = b1_ref[...]
    w2 = w2_ref[...]
    b2 = b2_ref[0, 0]

    def mlp(u):
        h1 = jnp.maximum(bill @ a1 + u @ a2 + b1, 0.0)
        return (h1 * w2).sum(axis=1, keepdims=True) + b2

    p = mlp(u1_ref[...])
    q = mlp(u2_ref[...])
    t = (jnp.maximum(p, 0.0) - p + jnp.log1p(jnp.exp(-jnp.abs(p)))
         + jnp.maximum(q, 0.0) + jnp.log1p(jnp.exp(-jnp.abs(q))))
    o_ref[0, 0] = jnp.sum(t) / (2.0 * p.shape[0])


def _head(bill, u1, u2, fc1_w, fc1_b, fc2_w, fc2_b):
    b, h = bill.shape
    a = fc1_w.T
    a1, a2 = a[:h], a[h:]
    return pl.pallas_call(
        _head_body,
        out_specs=pl.BlockSpec(memory_space=pltpu.SMEM),
        out_shape=jax.ShapeDtypeStruct((1, 1), jnp.float32),
    )(bill, u1, u2, a1, a2, fc1_b.reshape(1, h),
      fc2_w.reshape(1, h), fc2_b.reshape(1, 1))


def _edge_agg(y, src, dst, etype, n, r, h):
    flat_src = src * r + etype
    msgs = y.reshape(n * r, h)[flat_src]
    flat_dst = dst * r + etype
    acc = jnp.zeros((n * r, h), jnp.float32).at[flat_dst].add(msgs)
    cnt = jnp.zeros((n * r,), jnp.float32).at[flat_dst].add(1.0)
    return acc.reshape(n, r * h), cnt.reshape(n, r)


def kernel(users_feature, W1, root1, b1, W2, root2, b2, fc1_w, fc1_b,
           fc2_w, fc2_b, edge_index, edge_type, bill_id, user1_id, user2_id):
    n, d = users_feature.shape
    r, _, h = W1.shape
    src = edge_index[0].astype(jnp.int32)
    dst = edge_index[1].astype(jnp.int32)
    etype = edge_type.astype(jnp.int32)

    w1_cat = W1.transpose(1, 0, 2).reshape(d, r * h)
    w2_cat = W2.transpose(1, 0, 2).reshape(h, r * h)

    y1, xr1 = _proj(users_feature, w1_cat, root1, rows_blk=1000)
    acc1, cnt = _edge_agg(y1, src, dst, etype, n, r, h)
    hfeat = _combine(xr1, acc1, cnt, b1, do_relu=True, rows_blk=1000)

    y2, xr2 = _proj(hfeat, w2_cat, root2, rows_blk=1000)
    acc2, _ = _edge_agg(y2, src, dst, etype, n, r, h)
    nodes = _combine(xr2, acc2, cnt, b2, do_relu=False, rows_blk=1000)

    bill = nodes[bill_id]
    u1 = nodes[user1_id]
    u2 = nodes[user2_id]
    return _head(bill, u1, u2, fc1_w, fc1_b, fc2_w, fc2_b)[0, 0]
